# Initial kernel scaffold; baseline (speedup 1.0000x reference)
import functools

import jax
import jax.numpy as jnp
from jax import lax
from jax.experimental import pallas as pl
from jax.experimental.pallas import tpu as pltpu
from jax.experimental.pallas import tpu_sc as plsc

_TEMP = 0.07
_BASE_TEMP = 0.07

_N = 8192
_B = _N // 2
_D = 128
_K = 64
_NC = 2
_NS = 16
_NW = _NC * _NS
_A = _B // _NW
_CH = 2
_NCH = _A // _CH
_L = 16


_neg_cache = None


def _neg_table():
    global _neg_cache
    if _neg_cache is None:
        def build():
            eye = jnp.eye(_B, dtype=jnp.float32)
            weights = (jnp.tile(eye, (1, 2)) - 1.0) * -1.0 / (2 * _B - 2)
            gkey = jax.random.fold_in(jax.random.key(0), 1)
            gumbel = jax.random.gumbel(gkey, weights.shape, dtype=jnp.float32)
            logits = jnp.where(
                weights > 0, jnp.log(jnp.maximum(weights, 1e-30)), -jnp.inf
            )
            _, neg_inds = lax.top_k(logits + gumbel, _K)
            return neg_inds.astype(jnp.int32)

        _neg_cache = jax.jit(build)()
    return _neg_cache


def _sc_body(feat_hbm, nidx_hbm, posd_hbm, negsum_hbm,
             anch_v, posr_v, nidx_v, rows_v, posd_v, ns_v, gsem):
    wid = lax.axis_index("s") * _NC + lax.axis_index("c")
    base = wid * _A

    pltpu.sync_copy(feat_hbm.at[pl.ds(base, _A)], anch_v)
    pltpu.sync_copy(feat_hbm.at[pl.ds(_B + base, _A)], posr_v)
    pltpu.sync_copy(nidx_hbm.at[pl.ds(base * _K, _A * _K)], nidx_v)

    lanes = lax.iota(jnp.int32, _L)

    def issue(ci, buf):
        idx_view = nidx_v.at[pl.ds(ci * (_CH * _K), _CH * _K)]
        dst = rows_v.at[pl.ds(buf * (_CH * _K), _CH * _K)]
        return pltpu.async_copy(feat_hbm.at[idx_view], dst, gsem)

    def compute_chunk(ci, buf):
        for j in range(_CH):
            a = ci * _CH + j
            a_splat = jnp.full((_L,), a, jnp.int32)
            rowbase = buf * (_CH * _K) + j * _K
            rb = [jnp.full((_L,), rowbase + g * _L, jnp.int32) + lanes
                  for g in range(_K // _L)]

            def dbody(t, acc):
                accs = list(acc)
                for u in range(4):
                    d = t * 4 + u
                    colv = (lanes + d) & (_D - 1)
                    av = plsc.load_gather(anch_v, [a_splat, colv])
                    for g in range(_K // _L):
                        vals = plsc.load_gather(rows_v, [rb[g], colv])
                        diff = vals - av
                        accs[g] = accs[g] + diff * diff
                return tuple(accs)

            zero = jnp.zeros((_L,), jnp.float32)
            accs = lax.fori_loop(0, _D // 4, dbody, (zero,) * (_K // _L))

            nsv = jnp.zeros((_L,), jnp.float32)
            for g in range(_K // _L):
                p = jnp.clip(1.0 / (1.0 + accs[g]), 0.0001, 1.0)
                nsv = nsv + p
            ns_v[a] = jnp.sum(nsv)

            pacc = jnp.zeros((_L,), jnp.float32)
            for c in range(_D // _L):
                av8 = anch_v[a, pl.ds(c * _L, _L)]
                pv8 = posr_v[a, pl.ds(c * _L, _L)]
                t8 = av8 - pv8
                pacc = pacc + t8 * t8
            posd_v[a] = jnp.sum(pacc)

    def cbody(ci, carry):
        buf = ci & 1
        h = issue(ci, buf)

        @pl.when(ci > 0)
        def _():
            compute_chunk(ci - 1, 1 - buf)

        h.wait()
        return carry

    lax.fori_loop(0, _NCH, cbody, 0)
    compute_chunk(_NCH - 1, (_NCH - 1) & 1)

    pltpu.sync_copy(posd_v, posd_hbm.at[pl.ds(base, _A)])
    pltpu.sync_copy(ns_v, negsum_hbm.at[pl.ds(base, _A)])


_sc_distances = functools.partial(
    pl.kernel,
    out_type=[
        jax.ShapeDtypeStruct((_B,), jnp.float32),
        jax.ShapeDtypeStruct((_B,), jnp.float32),
    ],
    mesh=plsc.VectorSubcoreMesh(core_axis_name="c", subcore_axis_name="s"),
    scratch_types=[
        pltpu.VMEM((_A, _D), jnp.float32),
        pltpu.VMEM((_A, _D), jnp.float32),
        pltpu.VMEM((_A * _K,), jnp.int32),
        pltpu.VMEM((2 * _CH * _K, _D), jnp.float32),
        pltpu.VMEM((_A,), jnp.float32),
        pltpu.VMEM((_A,), jnp.float32),
        pltpu.SemaphoreType.DMA,
    ],
)(_sc_body)


def _loss_body(posd_ref, ns_ref, out_ref):
    p = jnp.clip(1.0 / (1.0 + posd_ref[...]), 0.0001, 1.0)
    li = -(_TEMP / _BASE_TEMP) * (jnp.log(p) - jnp.log(ns_ref[...]))
    out_ref[0, 0] = jnp.sum(li) / _B


def _tc_loss(posd, ns):
    return pl.pallas_call(
        _loss_body,
        out_shape=jax.ShapeDtypeStruct((1, 1), jnp.float32),
    )(posd.reshape(_NW, _A), ns.reshape(_NW, _A))


def kernel(features):
    nidx = _neg_table().reshape(-1)
    posd, negsum = _sc_distances(features, nidx)
    return _tc_loss(posd, negsum)[0, 0]


# speedup vs baseline: 1.0724x; 1.0724x over previous
import functools

import jax
import jax.numpy as jnp
from jax import lax
from jax.experimental import pallas as pl
from jax.experimental.pallas import tpu as pltpu
from jax.experimental.pallas import tpu_sc as plsc

_TEMP = 0.07
_BASE_TEMP = 0.07

_N = 8192
_B = _N // 2
_D = 128
_K = 64
_NC = 2
_NS = 16
_NW = _NC * _NS
_A = _B // _NW
_CH = 2
_NCH = _A // _CH
_L = 16


_neg_cache = None


def _neg_table():
    global _neg_cache
    if _neg_cache is None:
        def build():
            eye = jnp.eye(_B, dtype=jnp.float32)
            weights = (jnp.tile(eye, (1, 2)) - 1.0) * -1.0 / (2 * _B - 2)
            gkey = jax.random.fold_in(jax.random.key(0), 1)
            gumbel = jax.random.gumbel(gkey, weights.shape, dtype=jnp.float32)
            logits = jnp.where(
                weights > 0, jnp.log(jnp.maximum(weights, 1e-30)), -jnp.inf
            )
            _, neg_inds = lax.top_k(logits + gumbel, _K)
            return neg_inds.astype(jnp.int32)

        _neg_cache = jax.jit(build)()
    return _neg_cache


def _sc_body(feat_hbm, nidx_hbm, posd_hbm, negsum_hbm,
             anch_v, posr_v, nidx_v, rows_v, posd_v, ns_v, gsem):
    wid = lax.axis_index("s") * _NC + lax.axis_index("c")
    base = wid * _A

    pltpu.sync_copy(feat_hbm.at[pl.ds(base, _A)], anch_v)
    pltpu.sync_copy(feat_hbm.at[pl.ds(_B + base, _A)], posr_v)
    pltpu.sync_copy(nidx_hbm.at[pl.ds(base * _K, _A * _K)], nidx_v)

    lanes = lax.iota(jnp.int32, _L)

    def issue(ci, buf):
        idx_view = nidx_v.at[pl.ds(ci * (_CH * _K), _CH * _K)]
        dst = rows_v.at[pl.ds(buf * (_CH * _K), _CH * _K)]
        return pltpu.async_copy(feat_hbm.at[idx_view], dst, gsem)

    def compute_chunk(ci, buf):
        for j in range(_CH):
            a = ci * _CH + j
            a_splat = jnp.full((_L,), a, jnp.int32)
            rowbase = buf * (_CH * _K) + j * _K
            rb = [jnp.full((_L,), rowbase + g * _L, jnp.int32) + lanes
                  for g in range(_K // _L)]

            def dbody(t, acc):
                accs = list(acc)
                for u in range(4):
                    d = t * 4 + u
                    colv = (lanes + d) & (_D - 1)
                    av = plsc.load_gather(anch_v, [a_splat, colv])
                    for g in range(_K // _L):
                        vals = plsc.load_gather(rows_v, [rb[g], colv])
                        diff = vals - av
                        accs[g] = accs[g] + diff * diff
                return tuple(accs)

            zero = jnp.zeros((_L,), jnp.float32)
            accs = lax.fori_loop(0, _D // 4, dbody, (zero,) * (_K // _L))

            nsv = jnp.zeros((_L,), jnp.float32)
            for g in range(_K // _L):
                p = jnp.clip(1.0 / (1.0 + accs[g]), 0.0001, 1.0)
                nsv = nsv + p
            ns_v[a] = nsv

            pacc = jnp.zeros((_L,), jnp.float32)
            for c in range(_D // _L):
                av8 = anch_v[a, pl.ds(c * _L, _L)]
                pv8 = posr_v[a, pl.ds(c * _L, _L)]
                t8 = av8 - pv8
                pacc = pacc + t8 * t8
            posd_v[a] = pacc

    def cbody(ci, carry):
        buf = ci & 1
        h = issue(ci, buf)

        @pl.when(ci > 0)
        def _():
            compute_chunk(ci - 1, 1 - buf)

        h.wait()
        return carry

    lax.fori_loop(0, _NCH, cbody, 0)
    compute_chunk(_NCH - 1, (_NCH - 1) & 1)

    pltpu.sync_copy(posd_v, posd_hbm.at[pl.ds(base, _A)])
    pltpu.sync_copy(ns_v, negsum_hbm.at[pl.ds(base, _A)])


_sc_distances = functools.partial(
    pl.kernel,
    out_type=[
        jax.ShapeDtypeStruct((_B, _L), jnp.float32),
        jax.ShapeDtypeStruct((_B, _L), jnp.float32),
    ],
    mesh=plsc.VectorSubcoreMesh(core_axis_name="c", subcore_axis_name="s"),
    compiler_params=pltpu.CompilerParams(needs_layout_passes=False),
    scratch_types=[
        pltpu.VMEM((_A, _D), jnp.float32),
        pltpu.VMEM((_A, _D), jnp.float32),
        pltpu.VMEM((_A * _K,), jnp.int32),
        pltpu.VMEM((2 * _CH * _K, _D), jnp.float32),
        pltpu.VMEM((_A, _L), jnp.float32),
        pltpu.VMEM((_A, _L), jnp.float32),
        pltpu.SemaphoreType.DMA,
    ],
)(_sc_body)


def _loss_body(posd_ref, ns_ref, out_ref):
    posd = jnp.sum(posd_ref[...], axis=1)
    ns = jnp.sum(ns_ref[...], axis=1)
    p = jnp.clip(1.0 / (1.0 + posd), 0.0001, 1.0)
    li = -(_TEMP / _BASE_TEMP) * (jnp.log(p) - jnp.log(ns))
    out_ref[0, 0] = jnp.sum(li) / _B


def _tc_loss(posd, ns):
    return pl.pallas_call(
        _loss_body,
        out_shape=jax.ShapeDtypeStruct((1, 1), jnp.float32),
        out_specs=pl.BlockSpec(memory_space=pltpu.SMEM),
    )(posd, ns)


def kernel(features):
    nidx = _neg_table().reshape(-1)
    posd, negsum = _sc_distances(features, nidx)
    return _tc_loss(posd, negsum)[0, 0]


# speedup vs baseline: 128.7993x; 120.1001x over previous
import functools

import jax
import jax.numpy as jnp
from jax import lax
from jax.experimental import pallas as pl
from jax.experimental.pallas import tpu as pltpu
from jax.experimental.pallas import tpu_sc as plsc

_TEMP = 0.07
_BASE_TEMP = 0.07

_N = 8192
_B = _N // 2
_D = 128
_K = 64
_NC = 2
_NS = 16
_NW = _NC * _NS
_A = _B // _NW
_CH = 2
_NCH = _A // _CH
_L = 16


_neg_cache = None


def _neg_table():
    global _neg_cache
    if _neg_cache is None:
        def build():
            eye = jnp.eye(_B, dtype=jnp.float32)
            weights = (jnp.tile(eye, (1, 2)) - 1.0) * -1.0 / (2 * _B - 2)
            gkey = jax.random.fold_in(jax.random.key(0), 1)
            gumbel = jax.random.gumbel(gkey, weights.shape, dtype=jnp.float32)
            logits = jnp.where(
                weights > 0, jnp.log(jnp.maximum(weights, 1e-30)), -jnp.inf
            )
            _, neg_inds = lax.top_k(logits + gumbel, _K)
            return neg_inds.astype(jnp.int32)

        with jax.ensure_compile_time_eval():
            _neg_cache = jax.jit(build)()
    return _neg_cache


def _sc_body(feat_hbm, nidx_hbm, posd_hbm, negsum_hbm,
             anch_v, posr_v, nidx_v, rows_v, posd_v, ns_v, gsem):
    wid = lax.axis_index("s") * _NC + lax.axis_index("c")
    base = wid * _A

    pltpu.sync_copy(feat_hbm.at[pl.ds(base, _A)], anch_v)
    pltpu.sync_copy(feat_hbm.at[pl.ds(_B + base, _A)], posr_v)
    pltpu.sync_copy(nidx_hbm.at[pl.ds(base * _K, _A * _K)], nidx_v)

    lanes = lax.iota(jnp.int32, _L)

    def issue(ci, buf):
        idx_view = nidx_v.at[pl.ds(ci * (_CH * _K), _CH * _K)]
        dst = rows_v.at[pl.ds(buf * (_CH * _K), _CH * _K)]
        return pltpu.async_copy(feat_hbm.at[idx_view], dst, gsem)

    def compute_chunk(ci, buf):
        for j in range(_CH):
            a = ci * _CH + j
            a_splat = jnp.full((_L,), a, jnp.int32)
            rowbase = buf * (_CH * _K) + j * _K
            rb = [jnp.full((_L,), rowbase + g * _L, jnp.int32) + lanes
                  for g in range(_K // _L)]

            def dbody(t, acc):
                accs = list(acc)
                for u in range(4):
                    d = t * 4 + u
                    colv = (lanes + d) & (_D - 1)
                    av = plsc.load_gather(anch_v, [a_splat, colv])
                    for g in range(_K // _L):
                        vals = plsc.load_gather(rows_v, [rb[g], colv])
                        diff = vals - av
                        accs[g] = accs[g] + diff * diff
                return tuple(accs)

            zero = jnp.zeros((_L,), jnp.float32)
            accs = lax.fori_loop(0, _D // 4, dbody, (zero,) * (_K // _L))

            nsv = jnp.zeros((_L,), jnp.float32)
            for g in range(_K // _L):
                p = jnp.clip(1.0 / (1.0 + accs[g]), 0.0001, 1.0)
                nsv = nsv + p
            ns_v[a] = nsv

            pacc = jnp.zeros((_L,), jnp.float32)
            for c in range(_D // _L):
                av8 = anch_v[a, pl.ds(c * _L, _L)]
                pv8 = posr_v[a, pl.ds(c * _L, _L)]
                t8 = av8 - pv8
                pacc = pacc + t8 * t8
            posd_v[a] = pacc

    def cbody(ci, carry):
        buf = ci & 1
        h = issue(ci, buf)

        @pl.when(ci > 0)
        def _():
            compute_chunk(ci - 1, 1 - buf)

        h.wait()
        return carry

    lax.fori_loop(0, _NCH, cbody, 0)
    compute_chunk(_NCH - 1, (_NCH - 1) & 1)

    pltpu.sync_copy(posd_v, posd_hbm.at[pl.ds(base, _A)])
    pltpu.sync_copy(ns_v, negsum_hbm.at[pl.ds(base, _A)])


_sc_distances = functools.partial(
    pl.kernel,
    out_type=[
        jax.ShapeDtypeStruct((_B, _L), jnp.float32),
        jax.ShapeDtypeStruct((_B, _L), jnp.float32),
    ],
    mesh=plsc.VectorSubcoreMesh(core_axis_name="c", subcore_axis_name="s"),
    compiler_params=pltpu.CompilerParams(needs_layout_passes=False),
    scratch_types=[
        pltpu.VMEM((_A, _D), jnp.float32),
        pltpu.VMEM((_A, _D), jnp.float32),
        pltpu.VMEM((_A * _K,), jnp.int32),
        pltpu.VMEM((2 * _CH * _K, _D), jnp.float32),
        pltpu.VMEM((_A, _L), jnp.float32),
        pltpu.VMEM((_A, _L), jnp.float32),
        pltpu.SemaphoreType.DMA,
    ],
)(_sc_body)


def _loss_body(posd_ref, ns_ref, out_ref):
    posd = jnp.sum(posd_ref[...], axis=1)
    ns = jnp.sum(ns_ref[...], axis=1)
    p = jnp.clip(1.0 / (1.0 + posd), 0.0001, 1.0)
    li = -(_TEMP / _BASE_TEMP) * (jnp.log(p) - jnp.log(ns))
    out_ref[0, 0] = jnp.sum(li) / _B


def _tc_loss(posd, ns):
    return pl.pallas_call(
        _loss_body,
        out_shape=jax.ShapeDtypeStruct((1, 1), jnp.float32),
        out_specs=pl.BlockSpec(memory_space=pltpu.SMEM),
    )(posd, ns)


def kernel(features):
    nidx = _neg_table().reshape(-1)
    posd, negsum = _sc_distances(features, nidx)
    return _tc_loss(posd, negsum)[0, 0]


# speedup vs baseline: 153.7885x; 1.1940x over previous
import functools

import jax
import jax.numpy as jnp
from jax import lax
from jax.experimental import pallas as pl
from jax.experimental.pallas import tpu as pltpu
from jax.experimental.pallas import tpu_sc as plsc

_TEMP = 0.07
_BASE_TEMP = 0.07

_N = 8192
_B = _N // 2
_D = 128
_W = _D // 2
_K = 64
_NC = 2
_NS = 16
_NW = _NC * _NS
_A = _B // _NW
_CH = 2
_NCH = _A // _CH
_L = 16


_neg_cache = None


def _neg_table():
    global _neg_cache
    if _neg_cache is None:
        def build():
            eye = jnp.eye(_B, dtype=jnp.float32)
            weights = (jnp.tile(eye, (1, 2)) - 1.0) * -1.0 / (2 * _B - 2)
            gkey = jax.random.fold_in(jax.random.key(0), 1)
            gumbel = jax.random.gumbel(gkey, weights.shape, dtype=jnp.float32)
            logits = jnp.where(
                weights > 0, jnp.log(jnp.maximum(weights, 1e-30)), -jnp.inf
            )
            _, neg_inds = lax.top_k(logits + gumbel, _K)
            return neg_inds.astype(jnp.int32)

        with jax.ensure_compile_time_eval():
            _neg_cache = jax.jit(build)()
    return _neg_cache


def _pack_body(f_ref, out_ref):
    bf = f_ref[...].astype(jnp.bfloat16)
    lo = lax.bitcast_convert_type(bf[:, :_W], jnp.uint16).astype(jnp.uint32)
    hi = lax.bitcast_convert_type(bf[:, _W:], jnp.uint16).astype(jnp.uint32)
    out_ref[...] = lax.bitcast_convert_type(lo | (hi << 16), jnp.int32)


def _pack_features(features):
    return pl.pallas_call(
        _pack_body,
        out_shape=jax.ShapeDtypeStruct((_N, _W), jnp.int32),
    )(features)


def _sc_body(pk_hbm, nidx_hbm, negsum_hbm, apk_v, nidx_v, rows_v, ns_v, gsem):
    wid = lax.axis_index("s") * _NC + lax.axis_index("c")
    base = wid * _A

    pltpu.sync_copy(pk_hbm.at[pl.ds(base, _A)], apk_v)
    pltpu.sync_copy(nidx_hbm.at[pl.ds(base * _K, _A * _K)], nidx_v)

    lanes = lax.iota(jnp.int32, _L)
    himask = jnp.full((_L,), -65536, jnp.int32)

    def issue(ci, buf):
        idx_view = nidx_v.at[pl.ds(ci * (_CH * _K), _CH * _K)]
        dst = rows_v.at[pl.ds(buf * (_CH * _K), _CH * _K)]
        return pltpu.async_copy(pk_hbm.at[idx_view], dst, gsem)

    def compute_chunk(ci, buf):
        for j in range(_CH):
            a = ci * _CH + j
            a_splat = jnp.full((_L,), a, jnp.int32)
            rowbase = buf * (_CH * _K) + j * _K
            rb = [jnp.full((_L,), rowbase + g * _L, jnp.int32) + lanes
                  for g in range(_K // _L)]

            def dbody(t, acc):
                accs = list(acc)
                for u in range(2):
                    s = t * 2 + u
                    colv = (lanes + s) & (_W - 1)
                    apk = plsc.load_gather(apk_v, [a_splat, colv])
                    abf = plsc.bitcast(apk, jnp.bfloat16)
                    for g in range(_K // _L):
                        rpk = plsc.load_gather(rows_v, [rb[g], colv])
                        dbf = plsc.bitcast(rpk, jnp.bfloat16) - abf
                        d32 = plsc.bitcast(dbf, jnp.int32)
                        dlo = plsc.bitcast(d32 << 16, jnp.float32)
                        dhi = plsc.bitcast(d32 & himask, jnp.float32)
                        accs[g] = accs[g] + dlo * dlo + dhi * dhi
                return tuple(accs)

            zero = jnp.zeros((_L,), jnp.float32)
            accs = lax.fori_loop(0, _W // 2, dbody, (zero,) * (_K // _L))

            nsv = jnp.zeros((_L,), jnp.float32)
            for g in range(_K // _L):
                p = jnp.clip(1.0 / (1.0 + accs[g]), 0.0001, 1.0)
                nsv = nsv + p
            ns_v[a] = nsv

    def cbody(ci, carry):
        buf = ci & 1
        h = issue(ci, buf)

        @pl.when(ci > 0)
        def _():
            compute_chunk(ci - 1, 1 - buf)

        h.wait()
        return carry

    lax.fori_loop(0, _NCH, cbody, 0)
    compute_chunk(_NCH - 1, (_NCH - 1) & 1)

    pltpu.sync_copy(ns_v, negsum_hbm.at[pl.ds(base, _A)])


_sc_distances = functools.partial(
    pl.kernel,
    out_type=jax.ShapeDtypeStruct((_B, _L), jnp.float32),
    mesh=plsc.VectorSubcoreMesh(core_axis_name="c", subcore_axis_name="s"),
    compiler_params=pltpu.CompilerParams(needs_layout_passes=False, use_tc_tiling_on_sc=False),
    scratch_types=[
        pltpu.VMEM((_A, _W), jnp.int32),
        pltpu.VMEM((_A * _K,), jnp.int32),
        pltpu.VMEM((2 * _CH * _K, _W), jnp.int32),
        pltpu.VMEM((_A, _L), jnp.float32),
        pltpu.SemaphoreType.DMA,
    ],
)(_sc_body)


def _loss_body(f_ref, ns_ref, out_ref):
    diff = f_ref[: _B, :] - f_ref[_B:, :]
    posd = jnp.sum(diff * diff, axis=1)
    ns = jnp.sum(ns_ref[...], axis=1)
    p = jnp.clip(1.0 / (1.0 + posd), 0.0001, 1.0)
    li = -(_TEMP / _BASE_TEMP) * (jnp.log(p) - jnp.log(ns))
    out_ref[0, 0] = jnp.sum(li) / _B


def _tc_loss(features, ns):
    return pl.pallas_call(
        _loss_body,
        out_shape=jax.ShapeDtypeStruct((1, 1), jnp.float32),
        out_specs=pl.BlockSpec(memory_space=pltpu.SMEM),
    )(features, ns)


def kernel(features):
    nidx = _neg_table().reshape(-1)
    packed = _pack_features(features)
    negsum = _sc_distances(packed, nidx)
    return _tc_loss(features, negsum)[0, 0]


# speedup vs baseline: 187.3161x; 1.2180x over previous
import functools

import jax
import jax.numpy as jnp
from jax import lax
from jax.experimental import pallas as pl
from jax.experimental.pallas import tpu as pltpu
from jax.experimental.pallas import tpu_sc as plsc

_TEMP = 0.07
_BASE_TEMP = 0.07

_N = 8192
_B = _N // 2
_D = 128
_W = _D // 2
_K = 64
_NC = 2
_NS = 16
_NW = _NC * _NS
_A = _B // _NW
_CH = 4
_NCH = _A // _CH
_L = 16
_SEG = 128


_neg_cache = None


def _neg_table():
    global _neg_cache
    if _neg_cache is None:
        def build():
            eye = jnp.eye(_B, dtype=jnp.float32)
            weights = (jnp.tile(eye, (1, 2)) - 1.0) * -1.0 / (2 * _B - 2)
            gkey = jax.random.fold_in(jax.random.key(0), 1)
            gumbel = jax.random.gumbel(gkey, weights.shape, dtype=jnp.float32)
            logits = jnp.where(
                weights > 0, jnp.log(jnp.maximum(weights, 1e-30)), -jnp.inf
            )
            _, neg_inds = lax.top_k(logits + gumbel, _K)
            return neg_inds.astype(jnp.int32)

        with jax.ensure_compile_time_eval():
            _neg_cache = jax.jit(build)()
    return _neg_cache


def _pack_body(f_ref, out_ref):
    bf = f_ref[...].astype(jnp.bfloat16)
    lo = lax.bitcast_convert_type(bf[:, :_W], jnp.uint16).astype(jnp.uint32)
    hi = lax.bitcast_convert_type(bf[:, _W:], jnp.uint16).astype(jnp.uint32)
    out_ref[...] = lax.bitcast_convert_type(lo | (hi << 16), jnp.int32)


def _pack_features(features):
    return pl.pallas_call(
        _pack_body,
        out_shape=jax.ShapeDtypeStruct((_N, _W), jnp.int32),
    )(features)


def _sc_body(pk_hbm, nidx_hbm, negsum_hbm, apk_v, nidx_v, rows_v, ns_v, gsem):
    wid = lax.axis_index("s") * _NC + lax.axis_index("c")
    base = wid * _A

    pltpu.sync_copy(pk_hbm.at[pl.ds(base, _A)], apk_v)
    pltpu.sync_copy(nidx_hbm.at[pl.ds(base * _K, _A * _K)], nidx_v)

    lanes = lax.iota(jnp.int32, _L)
    himask = jnp.full((_L,), -65536, jnp.int32)

    def issue(ci, buf):
        hs = []
        for seg in range(_CH * _K // _SEG):
            idx_view = nidx_v.at[pl.ds(ci * (_CH * _K) + seg * _SEG, _SEG)]
            dst = rows_v.at[pl.ds(buf * (_CH * _K) + seg * _SEG, _SEG)]
            hs.append(pltpu.async_copy(pk_hbm.at[idx_view], dst, gsem))
        return hs

    def compute_chunk(ci, buf):
        for j in range(_CH):
            a = ci * _CH + j
            a_splat = jnp.full((_L,), a, jnp.int32)
            rowbase = buf * (_CH * _K) + j * _K
            rb = [jnp.full((_L,), rowbase + g * _L, jnp.int32) + lanes
                  for g in range(_K // _L)]

            def dbody(t, acc):
                accs = list(acc)
                accb = [jnp.zeros((2 * _L,), jnp.bfloat16)
                        for _ in range(_K // _L)]
                for u in range(8):
                    s = t * 8 + u
                    colv = (lanes + s) & (_W - 1)
                    apk = plsc.load_gather(apk_v, [a_splat, colv])
                    abf = plsc.bitcast(apk, jnp.bfloat16)
                    for g in range(_K // _L):
                        rpk = plsc.load_gather(rows_v, [rb[g], colv])
                        dbf = plsc.bitcast(rpk, jnp.bfloat16) - abf
                        accb[g] = accb[g] + dbf * dbf
                for g in range(_K // _L):
                    b32 = plsc.bitcast(accb[g], jnp.int32)
                    blo = plsc.bitcast(b32 << 16, jnp.float32)
                    bhi = plsc.bitcast(b32 & himask, jnp.float32)
                    accs[g] = accs[g] + blo + bhi
                return tuple(accs)

            zero = jnp.zeros((_L,), jnp.float32)
            accs = lax.fori_loop(0, _W // 8, dbody, (zero,) * (_K // _L))

            nsv = jnp.zeros((_L,), jnp.float32)
            for g in range(_K // _L):
                p = jnp.clip(1.0 / (1.0 + accs[g]), 0.0001, 1.0)
                nsv = nsv + p
            ns_v[a] = nsv

    def cbody(ci, carry):
        buf = ci & 1
        hs = issue(ci, buf)

        @pl.when(ci > 0)
        def _():
            compute_chunk(ci - 1, 1 - buf)

        for h in hs:
            h.wait()
        return carry

    lax.fori_loop(0, _NCH, cbody, 0)
    compute_chunk(_NCH - 1, (_NCH - 1) & 1)

    pltpu.sync_copy(ns_v, negsum_hbm.at[pl.ds(base, _A)])


_sc_distances = functools.partial(
    pl.kernel,
    out_type=jax.ShapeDtypeStruct((_B, _L), jnp.float32),
    mesh=plsc.VectorSubcoreMesh(core_axis_name="c", subcore_axis_name="s"),
    compiler_params=pltpu.CompilerParams(needs_layout_passes=False, use_tc_tiling_on_sc=False),
    scratch_types=[
        pltpu.VMEM((_A, _W), jnp.int32),
        pltpu.VMEM((_A * _K,), jnp.int32),
        pltpu.VMEM((2 * _CH * _K, _W), jnp.int32),
        pltpu.VMEM((_A, _L), jnp.float32),
        pltpu.SemaphoreType.DMA,
    ],
)(_sc_body)


def _loss_body(f_ref, ns_ref, out_ref):
    diff = f_ref[: _B, :] - f_ref[_B:, :]
    posd = jnp.sum(diff * diff, axis=1)
    ns = jnp.sum(ns_ref[...], axis=1)
    p = jnp.clip(1.0 / (1.0 + posd), 0.0001, 1.0)
    li = -(_TEMP / _BASE_TEMP) * (jnp.log(p) - jnp.log(ns))
    out_ref[0, 0] = jnp.sum(li) / _B


def _tc_loss(features, ns):
    return pl.pallas_call(
        _loss_body,
        out_shape=jax.ShapeDtypeStruct((1, 1), jnp.float32),
        out_specs=pl.BlockSpec(memory_space=pltpu.SMEM),
    )(features, ns)


def kernel(features):
    nidx = _neg_table().reshape(-1)
    packed = _pack_features(features)
    negsum = _sc_distances(packed, nidx)
    return _tc_loss(features, negsum)[0, 0]


# speedup vs baseline: 202.8502x; 1.0829x over previous
import functools

import jax
import jax.numpy as jnp
from jax import lax
from jax.experimental import pallas as pl
from jax.experimental.pallas import tpu as pltpu
from jax.experimental.pallas import tpu_sc as plsc

_TEMP = 0.07
_BASE_TEMP = 0.07

_N = 8192
_B = _N // 2
_D = 128
_W = _D // 2
_K = 64
_NC = 2
_NS = 16
_NW = _NC * _NS
_A = _B // _NW
_CH = 8
_NCH = _A // _CH
_L = 16
_SEG = 128


_neg_cache = None


def _neg_table():
    global _neg_cache
    if _neg_cache is None:
        def build():
            eye = jnp.eye(_B, dtype=jnp.float32)
            weights = (jnp.tile(eye, (1, 2)) - 1.0) * -1.0 / (2 * _B - 2)
            gkey = jax.random.fold_in(jax.random.key(0), 1)
            gumbel = jax.random.gumbel(gkey, weights.shape, dtype=jnp.float32)
            logits = jnp.where(
                weights > 0, jnp.log(jnp.maximum(weights, 1e-30)), -jnp.inf
            )
            _, neg_inds = lax.top_k(logits + gumbel, _K)
            return neg_inds.astype(jnp.int32)

        with jax.ensure_compile_time_eval():
            _neg_cache = jax.jit(build)()
    return _neg_cache


def _pack_body(f_ref, out_ref):
    bf = f_ref[...].astype(jnp.bfloat16)
    lo = lax.bitcast_convert_type(bf[:, :_W], jnp.uint16).astype(jnp.uint32)
    hi = lax.bitcast_convert_type(bf[:, _W:], jnp.uint16).astype(jnp.uint32)
    out_ref[...] = lax.bitcast_convert_type(lo | (hi << 16), jnp.int32)


def _pack_features(features):
    return pl.pallas_call(
        _pack_body,
        out_shape=jax.ShapeDtypeStruct((_N, _W), jnp.int32),
    )(features)


def _sc_body(pk_hbm, nidx_hbm, negsum_hbm, apk_v, nidx_v, rows_v, ns_v, gsem):
    wid = lax.axis_index("s") * _NC + lax.axis_index("c")
    base = wid * _A

    pltpu.sync_copy(pk_hbm.at[pl.ds(base, _A)], apk_v)
    pltpu.sync_copy(nidx_hbm.at[pl.ds(base * _K, _A * _K)], nidx_v)

    lanes = lax.iota(jnp.int32, _L)
    himask = jnp.full((_L,), -65536, jnp.int32)

    def issue(ci, buf):
        hs = []
        for seg in range(_CH * _K // _SEG):
            idx_view = nidx_v.at[pl.ds(ci * (_CH * _K) + seg * _SEG, _SEG)]
            dst = rows_v.at[pl.ds(buf * (_CH * _K) + seg * _SEG, _SEG)]
            hs.append(pltpu.async_copy(pk_hbm.at[idx_view], dst, gsem))
        return hs

    def compute_chunk(ci, buf):
        for j in range(_CH):
            a = ci * _CH + j
            a_splat = jnp.full((_L,), a, jnp.int32)
            rowbase = buf * (_CH * _K) + j * _K
            rb = [jnp.full((_L,), rowbase + g * _L, jnp.int32) + lanes
                  for g in range(_K // _L)]

            def dbody(t, acc):
                accs = list(acc)
                accb = [jnp.zeros((2 * _L,), jnp.bfloat16)
                        for _ in range(_K // _L)]
                for u in range(8):
                    s = t * 8 + u
                    colv = (lanes + s) & (_W - 1)
                    apk = plsc.load_gather(apk_v, [a_splat, colv])
                    abf = plsc.bitcast(apk, jnp.bfloat16)
                    for g in range(_K // _L):
                        rpk = plsc.load_gather(rows_v, [rb[g], colv])
                        dbf = plsc.bitcast(rpk, jnp.bfloat16) - abf
                        accb[g] = accb[g] + dbf * dbf
                for g in range(_K // _L):
                    b32 = plsc.bitcast(accb[g], jnp.int32)
                    blo = plsc.bitcast(b32 << 16, jnp.float32)
                    bhi = plsc.bitcast(b32 & himask, jnp.float32)
                    accs[g] = accs[g] + blo + bhi
                return tuple(accs)

            zero = jnp.zeros((_L,), jnp.float32)
            accs = lax.fori_loop(0, _W // 8, dbody, (zero,) * (_K // _L))

            nsv = jnp.zeros((_L,), jnp.float32)
            for g in range(_K // _L):
                p = jnp.clip(1.0 / (1.0 + accs[g]), 0.0001, 1.0)
                nsv = nsv + p
            ns_v[a >> 3, pl.ds((a & 7) * _L, _L)] = nsv

    def cbody(ci, carry):
        buf = ci & 1
        hs = issue(ci, buf)

        @pl.when(ci > 0)
        def _():
            compute_chunk(ci - 1, 1 - buf)

        for h in hs:
            h.wait()
        return carry

    lax.fori_loop(0, _NCH, cbody, 0)
    compute_chunk(_NCH - 1, (_NCH - 1) & 1)

    pltpu.sync_copy(ns_v, negsum_hbm.at[pl.ds(wid * (_A * _L // 128), _A * _L // 128)])


_sc_distances = functools.partial(
    pl.kernel,
    out_type=jax.ShapeDtypeStruct((_B * _L // 128, 128), jnp.float32),
    mesh=plsc.VectorSubcoreMesh(core_axis_name="c", subcore_axis_name="s"),
    compiler_params=pltpu.CompilerParams(needs_layout_passes=False, use_tc_tiling_on_sc=False),
    scratch_types=[
        pltpu.VMEM((_A, _W), jnp.int32),
        pltpu.VMEM((_A * _K,), jnp.int32),
        pltpu.VMEM((2 * _CH * _K, _W), jnp.int32),
        pltpu.VMEM((_A * _L // 128, 128), jnp.float32),
        pltpu.SemaphoreType.DMA,
    ],
)(_sc_body)


def _loss_body(f_ref, ns_ref, out_ref):
    diff = f_ref[: _B, :] - f_ref[_B:, :]
    posd = jnp.sum(diff * diff, axis=1)
    p = jnp.clip(1.0 / (1.0 + posd), 0.0001, 1.0)
    grp = (lax.broadcasted_iota(jnp.int32, (128, 8), 0) // _L
           == lax.broadcasted_iota(jnp.int32, (128, 8), 1)).astype(jnp.float32)
    ns8 = jnp.dot(ns_ref[...], grp, preferred_element_type=jnp.float32)
    li = jnp.sum(jnp.log(p)) - jnp.sum(jnp.log(ns8))
    out_ref[0, 0] = -(_TEMP / _BASE_TEMP) * li / _B


def _tc_loss(features, ns):
    return pl.pallas_call(
        _loss_body,
        out_shape=jax.ShapeDtypeStruct((1, 1), jnp.float32),
        out_specs=pl.BlockSpec(memory_space=pltpu.SMEM),
    )(features, ns)


def kernel(features):
    nidx = _neg_table().reshape(-1)
    packed = _pack_features(features)
    negsum = _sc_distances(packed, nidx)
    return _tc_loss(features, negsum)[0, 0]
